# baseline (speedup 1.0000x reference)
import functools

import jax
import jax.numpy as jnp
import numpy as np
from jax import lax
from jax.experimental import pallas as pl
from jax.experimental.pallas import tpu as pltpu
from jax.experimental.pallas import tpu_sc as plsc

N_FIELDS = 26
VOCAB_PER_FIELD = 100000
BATCH = 4096
EMBED_DIM = 64
TOTAL_ROWS = N_FIELDS * VOCAB_PER_FIELD
N_IDX = BATCH * N_FIELDS

GAMMA = 2000.0
PRETRAIN_EPOCH = 5
_TEMP = float(GAMMA ** (1.0 / (PRETRAIN_EPOCH - 1)))
_SCALING = float(1.0 + np.exp(-0.5))

_NC = 2
_NS = 16
_NW = _NC * _NS
_LANES = 16

_SLABS = [
    (0, 600000, 0, 6, 0),
    (600000, 600000, 6, 6, 0),
    (1200000, 600000, 12, 6, 0),
    (1800000, 600000, 18, 6, 0),
    (2399360, 200640, 24, 2, 640),
]


def _slab_copy_body(t_hbm, o0, o1, o2, o3, o4, sem):
    outs = (o0, o1, o2, o3, o4)
    descs = []
    for j, (r0, nrows, _, _, _) in enumerate(_SLABS):
        d = pltpu.make_async_copy(t_hbm.at[pl.ds(r0, nrows), :], outs[j], sem)
        d.start()
        descs.append(d)
    for d in descs:
        d.wait()


_slab_copy = pl.pallas_call(
    _slab_copy_body,
    in_specs=[pl.BlockSpec(memory_space=pl.ANY)],
    out_specs=[pl.BlockSpec(memory_space=pl.ANY)] * len(_SLABS),
    out_shape=[jax.ShapeDtypeStruct((nrows, 1), jnp.float32)
               for _, nrows, _, _, _ in _SLABS],
    scratch_shapes=[pltpu.SemaphoreType.DMA],
)

_VECS = BATCH // _LANES


def _sc_gather_body(raw_hbm, t0, t1, t2, t3, t4, out_hbm, idx_v, rows_v, sem):
    f = lax.axis_index("s") * _NC + lax.axis_index("c")

    @pl.when(f < N_FIELDS)
    def _():
        base = f * BATCH
        pltpu.sync_copy(raw_hbm.at[pl.ds(base, BATCH)], idx_v)

        slab = jnp.minimum(f // 6, 4)
        local = f - slab * 6
        off = jnp.where(f >= 24, local * VOCAB_PER_FIELD + 640,
                        local * VOCAB_PER_FIELD).astype(jnp.int32)

        @pl.loop(0, _VECS, unroll=8)
        def _(i):
            s = pl.ds(i * _LANES, _LANES)
            idx_v[s] = idx_v[s] + off

        for j, tab in enumerate((t0, t1, t2, t3, t4)):
            lo = j * 6

            @pl.when((f >= lo) & (f < min(lo + 6, N_FIELDS)))
            def _(tab=tab):
                pltpu.async_copy(tab.at[idx_v], rows_v, sem).wait()

        pltpu.sync_copy(rows_v, out_hbm.at[pl.ds(base, BATCH)])


_sc_gather = functools.partial(
    pl.kernel,
    out_type=jax.ShapeDtypeStruct((N_IDX,), jnp.float32),
    mesh=plsc.VectorSubcoreMesh(
        core_axis_name="c", subcore_axis_name="s", num_cores=_NC,
        num_subcores=_NS,
    ),
    scratch_types=[
        pltpu.VMEM((BATCH,), jnp.int32),
        pltpu.VMEM((BATCH,), jnp.float32),
        pltpu.SemaphoreType.DMA,
    ],
)(_sc_gather_body)

_F_BLK = 2


def _tc_mul_body(x_ref, mw_ref, o_ref):
    gate = _SCALING * jax.nn.sigmoid(_TEMP * mw_ref[...])
    o_ref[...] = x_ref[...] * gate.reshape(_F_BLK, 1, BATCH)


_tc_mul = pl.pallas_call(
    _tc_mul_body,
    grid=(N_FIELDS // _F_BLK,),
    in_specs=[
        pl.BlockSpec((_F_BLK, EMBED_DIM, BATCH), lambda f: (f, 0, 0)),
        pl.BlockSpec((_F_BLK * BATCH,), lambda f: (f,)),
    ],
    out_specs=pl.BlockSpec((_F_BLK, EMBED_DIM, BATCH), lambda f: (f, 0, 0)),
    out_shape=jax.ShapeDtypeStruct((N_FIELDS, EMBED_DIM, BATCH), jnp.float32),
)


def kernel(x, current_epoch, current_step, raw_data, mask_weight):
    xt = jnp.transpose(x, (1, 2, 0))
    raw_flat = jnp.transpose(raw_data, (1, 0)).astype(jnp.int32).reshape(-1)
    slabs = [s.reshape(-1) for s in _slab_copy(mask_weight)]
    mw_flat = _sc_gather(raw_flat, *slabs)
    out_t = _tc_mul(xt, mw_flat)
    return jnp.transpose(out_t, (2, 0, 1))


# speedup vs baseline: 685.1919x; 685.1919x over previous
import functools

import jax
import jax.numpy as jnp
import numpy as np
from jax import lax
from jax.experimental import pallas as pl
from jax.experimental.pallas import tpu as pltpu
from jax.experimental.pallas import tpu_sc as plsc

N_FIELDS = 26
VOCAB_PER_FIELD = 100000
BATCH = 4096
EMBED_DIM = 64
TOTAL_ROWS = N_FIELDS * VOCAB_PER_FIELD
N_IDX = BATCH * N_FIELDS

GAMMA = 2000.0
PRETRAIN_EPOCH = 5
_TEMP = float(GAMMA ** (1.0 / (PRETRAIN_EPOCH - 1)))
_SCALING = float(1.0 + np.exp(-0.5))

_NC = 2
_NS = 16
_NW = _NC * _NS
_LANES = 16

_SLABS = [
    (0, 600000, 0, 6, 0),
    (600000, 600000, 6, 6, 0),
    (1200000, 600000, 12, 6, 0),
    (1800000, 600000, 18, 6, 0),
    (2399360, 200640, 24, 2, 640),
]


_VECS = BATCH // _LANES


def _sc_gather_body(raw_hbm, t0, t1, t2, t3, t4, out_hbm, idx_v, rows_v, sem):
    f = lax.axis_index("s") * _NC + lax.axis_index("c")

    @pl.when(f < N_FIELDS)
    def _():
        base = f * BATCH
        pltpu.sync_copy(raw_hbm.at[pl.ds(base, BATCH)], idx_v)

        slab = jnp.minimum(f // 6, 4)
        local = f - slab * 6
        off = jnp.where(f >= 24, local * VOCAB_PER_FIELD + 640,
                        local * VOCAB_PER_FIELD).astype(jnp.int32)

        @pl.loop(0, _VECS, unroll=8)
        def _(i):
            s = pl.ds(i * _LANES, _LANES)
            idx_v[s] = idx_v[s] + off

        for j, tab in enumerate((t0, t1, t2, t3, t4)):
            lo = j * 6

            @pl.when((f >= lo) & (f < min(lo + 6, N_FIELDS)))
            def _(tab=tab):
                pltpu.async_copy(tab.at[idx_v], rows_v, sem).wait()

        pltpu.sync_copy(rows_v, out_hbm.at[pl.ds(base, BATCH)])


_sc_gather = functools.partial(
    pl.kernel,
    out_type=jax.ShapeDtypeStruct((N_IDX,), jnp.float32),
    mesh=plsc.VectorSubcoreMesh(
        core_axis_name="c", subcore_axis_name="s", num_cores=_NC,
        num_subcores=_NS,
    ),
    scratch_types=[
        pltpu.VMEM((BATCH,), jnp.int32),
        pltpu.VMEM((BATCH,), jnp.float32),
        pltpu.SemaphoreType.DMA,
    ],
)(_sc_gather_body)

_F_BLK = 2


def _tc_mul_body(x_ref, mw_ref, o_ref):
    gate = _SCALING * jax.nn.sigmoid(_TEMP * mw_ref[...])
    o_ref[...] = x_ref[...] * gate.reshape(_F_BLK, 1, BATCH)


_tc_mul = pl.pallas_call(
    _tc_mul_body,
    grid=(N_FIELDS // _F_BLK,),
    in_specs=[
        pl.BlockSpec((_F_BLK, EMBED_DIM, BATCH), lambda f: (f, 0, 0)),
        pl.BlockSpec((_F_BLK * BATCH,), lambda f: (f,)),
    ],
    out_specs=pl.BlockSpec((_F_BLK, EMBED_DIM, BATCH), lambda f: (f, 0, 0)),
    out_shape=jax.ShapeDtypeStruct((N_FIELDS, EMBED_DIM, BATCH), jnp.float32),
)


def kernel(x, current_epoch, current_step, raw_data, mask_weight):
    xt = jnp.transpose(x, (1, 2, 0))
    raw_flat = jnp.transpose(raw_data, (1, 0)).astype(jnp.int32).reshape(-1)
    slabs = [mask_weight[r0:r0 + nrows].reshape(-1)
             for r0, nrows, _, _, _ in _SLABS]
    mw_flat = _sc_gather(raw_flat, *slabs)
    out_t = _tc_mul(xt, mw_flat)
    return jnp.transpose(out_t, (2, 0, 1))
